# baseline (speedup 1.0000x reference)
import jax
import jax.numpy as jnp
from jax import lax
from jax.experimental import pallas as pl
from jax.experimental.pallas import tpu as pltpu
from jax.experimental.pallas import tpu_sc as plsc

B = 16384
K = 50
D = 64
V = 1000000
NC = 2
NS = 16
NW = NC * NS
SENT_PER_W = B // NW
S = 16
CHUNKS = SENT_PER_W // S
ROWS = S * K
LANES = 16
DV = D // LANES
KP = 64
KG = (K + LANES - 1) // LANES


def _sc_body(ids_hbm, idsp_hbm, w_hbm, table_hbm, out_hbm,
             idx_v, gidx_v, ip_v, w_v, rows_v, out_v, sem):
    wid = lax.axis_index("s") * NC + lax.axis_index("c")
    base_s = wid * SENT_PER_W

    def chunk_body(c, carry):
        s0 = pl.multiple_of(base_s + c * S, S)
        f0 = pl.multiple_of(s0 * K, S * K)
        p0 = pl.multiple_of(s0 * KP, S * KP)
        pltpu.sync_copy(ids_hbm.at[pl.ds(f0, ROWS)], idx_v)
        pltpu.sync_copy(idsp_hbm.at[pl.ds(p0, S * KP)], ip_v)
        pltpu.sync_copy(w_hbm.at[pl.ds(p0, S * KP)], w_v)
        for g in range(ROWS // LANES):
            sl = pl.ds(g * LANES, LANES)
            gidx_v[sl] = lax.shift_right_logical(idx_v[sl], 1)
        pltpu.async_copy(table_hbm.at[gidx_v], rows_v, sem).wait()

        def sent_body(s, carry2):
            r0 = s * K
            wb = s * KP
            zero = jnp.zeros((LANES,), jnp.float32)
            accs = [zero] * DV
            for g in range(KG):
                cnt = min(LANES, K - g * LANES)
                w16 = w_v[pl.ds(wb + g * LANES, LANES)]
                o16 = lax.shift_left(
                    lax.bitwise_and(ip_v[pl.ds(wb + g * LANES, LANES)], 1), 6)
                for j in range(cnt):
                    wv = lax.broadcast(w16[j], (LANES,))
                    off = o16[j]
                    fi = r0 + g * LANES + j
                    for d in range(DV):
                        accs[d] = accs[d] + wv * rows_v[fi, pl.ds(off + d * LANES, LANES)]
            inv_k = jnp.float32(1.0 / K)
            orow = lax.shift_right_logical(s, 1)
            obase = lax.shift_left(lax.bitwise_and(s, 1), 6)
            for d in range(DV):
                out_v[orow, pl.ds(obase + d * LANES, LANES)] = accs[d] * inv_k
            return carry2

        lax.fori_loop(0, S, sent_body, 0)
        pltpu.sync_copy(out_v, out_hbm.at[pl.ds(pl.multiple_of(s0 // 2, S // 2), S // 2)])
        return carry

    lax.fori_loop(0, CHUNKS, chunk_body, 0)


@jax.jit
def kernel(token_ids, weights, table):
    ids = token_ids.astype(jnp.int32)
    ids_flat = ids.reshape(-1)
    ids_pad = jnp.pad(ids, ((0, 0), (0, KP - K))).reshape(-1)
    w_flat = jnp.pad(weights, ((0, 0), (0, KP - K))).reshape(-1)
    table2 = table.reshape(V // 2, 2 * D)
    mesh = plsc.VectorSubcoreMesh(core_axis_name="c", subcore_axis_name="s")
    out2 = pl.kernel(
        _sc_body,
        out_type=jax.ShapeDtypeStruct((B // 2, 2 * D), jnp.float32),
        mesh=mesh,
        scratch_types=[
            pltpu.VMEM((ROWS,), jnp.int32),
            pltpu.VMEM((ROWS,), jnp.int32),
            pltpu.VMEM((S * KP,), jnp.int32),
            pltpu.VMEM((S * KP,), jnp.float32),
            pltpu.VMEM((ROWS, 2 * D), jnp.float32),
            pltpu.VMEM((S // 2, 2 * D), jnp.float32),
            pltpu.SemaphoreType.DMA,
        ],
    )(ids_flat, ids_pad, w_flat, table2)
    return out2.reshape(B, D)


# speedup vs baseline: 1.1139x; 1.1139x over previous
import jax
import jax.numpy as jnp
from jax import lax
from jax.experimental import pallas as pl
from jax.experimental.pallas import tpu as pltpu
from jax.experimental.pallas import tpu_sc as plsc

B = 16384
K = 50
D = 64
V = 1000000
NC = 2
NS = 16
NW = NC * NS
SENT_PER_W = B // NW
S = 16
CHUNKS = SENT_PER_W // S
ROWS = S * K
LANES = 16
DV = D // LANES
KP = 64
KG = (K + LANES - 1) // LANES


def _sc_body(ids_hbm, w_hbm, table_hbm, out_hbm, idx_v, w_v, rows_v, out_v, sem):
    wid = lax.axis_index("s") * NC + lax.axis_index("c")
    base_s = wid * SENT_PER_W

    def chunk_body(c, carry):
        s0 = pl.multiple_of(base_s + c * S, S)
        f0 = pl.multiple_of(s0 * K, S * K)
        p0 = pl.multiple_of(s0 * KP, S * KP)
        pltpu.sync_copy(ids_hbm.at[pl.ds(f0, ROWS)], idx_v)
        pltpu.sync_copy(w_hbm.at[pl.ds(p0, S * KP)], w_v)
        pltpu.async_copy(table_hbm.at[idx_v], rows_v, sem).wait()

        def sent_body(s, carry2):
            r0 = s * K
            wb = s * KP
            zero = jnp.zeros((LANES,), jnp.float32)
            accs = [zero] * DV
            for g in range(KG):
                cnt = min(LANES, K - g * LANES)
                w16 = w_v[pl.ds(wb + g * LANES, LANES)]
                for j in range(cnt):
                    wv = lax.broadcast(w16[j], (LANES,))
                    fi = r0 + g * LANES + j
                    for d in range(DV):
                        accs[d] = accs[d] + wv * rows_v[fi, pl.ds(d * LANES, LANES)]
            inv_k = jnp.float32(1.0 / K)
            orow = lax.shift_right_logical(s, 1)
            obase = lax.shift_left(lax.bitwise_and(s, 1), 6)
            for d in range(DV):
                out_v[orow, pl.ds(obase + d * LANES, LANES)] = accs[d] * inv_k
            return carry2

        lax.fori_loop(0, S, sent_body, 0)
        pltpu.sync_copy(out_v, out_hbm.at[pl.ds(pl.multiple_of(s0 // 2, S // 2), S // 2)])
        return carry

    lax.fori_loop(0, CHUNKS, chunk_body, 0)


@jax.jit
def kernel(token_ids, weights, table):
    ids_flat = token_ids.astype(jnp.int32).reshape(-1)
    w_flat = jnp.pad(weights, ((0, 0), (0, KP - K))).reshape(-1)
    table_pad = jnp.pad(table, ((0, 0), (0, 2 * D - D)))
    mesh = plsc.VectorSubcoreMesh(core_axis_name="c", subcore_axis_name="s")
    out2 = pl.kernel(
        _sc_body,
        out_type=jax.ShapeDtypeStruct((B // 2, 2 * D), jnp.float32),
        mesh=mesh,
        scratch_types=[
            pltpu.VMEM((ROWS,), jnp.int32),
            pltpu.VMEM((S * KP,), jnp.float32),
            pltpu.VMEM((ROWS, 2 * D), jnp.float32),
            pltpu.VMEM((S // 2, 2 * D), jnp.float32),
            pltpu.SemaphoreType.DMA,
        ],
    )(ids_flat, w_flat, table_pad)
    return out2.reshape(B, D)
